# Initial kernel scaffold; baseline (speedup 1.0000x reference)
import jax
import jax.numpy as jnp
from jax.experimental import pallas as pl


def kernel(x, table):
    raise NotImplementedError("write your pallas kernel here")


# speedup vs baseline: 1.9749x; 1.9749x over previous
import jax
import jax.numpy as jnp
from jax.experimental import pallas as pl
from jax.experimental.pallas import tpu as pltpu

SEQ_LEN = 8192
BATCH = 2
D_MODEL = 1024
BLOCK_S = 512


def _add_body(x_ref, t_ref, o_ref):
    o_ref[...] = x_ref[...] + t_ref[...][:, None, :]


def kernel(x, table):
    grid = (SEQ_LEN // BLOCK_S,)
    return pl.pallas_call(
        _add_body,
        grid=grid,
        in_specs=[
            pl.BlockSpec((BLOCK_S, BATCH, D_MODEL), lambda i: (i, 0, 0)),
            pl.BlockSpec((BLOCK_S, D_MODEL), lambda i: (i, 0)),
        ],
        out_specs=pl.BlockSpec((BLOCK_S, BATCH, D_MODEL), lambda i: (i, 0, 0)),
        out_shape=jax.ShapeDtypeStruct((SEQ_LEN, BATCH, D_MODEL), jnp.float32),
    )(x, table)
